# baseline (device time: 104191 ns/iter reference)
import jax
import jax.numpy as jnp
from jax import lax
from jax.experimental import pallas as pl
from jax.experimental.pallas import tpu as pltpu

N_DEV = 8
N_PIECE = 4

_DeviceIdType = getattr(pl, "DeviceIdType", None) or pltpu.DeviceIdType
MESH = _DeviceIdType.MESH
_sem_signal = getattr(pl, "semaphore_signal", None) or pltpu.semaphore_signal
_sem_wait = getattr(pl, "semaphore_wait", None) or pltpu.semaphore_wait
_CompilerParams = getattr(pltpu, "CompilerParams", None) or pltpu.TPUCompilerParams


def kernel(x, w_mat, scale_x, scale_w):
    m_per, k = x.shape
    _, n_per = w_mat.shape
    half = m_per // 2
    piece = half // N_PIECE

    x8 = x.astype(jnp.float8_e4m3fn)
    w8 = w_mat.astype(jnp.float8_e5m2)
    scale = (scale_x.reshape(-1)[:1] * scale_w.reshape(-1)[:1]).astype(jnp.float32)

    def body(x_ref, w_ref, s_ref, out_ref, xg_ref,
             cw_send, cw_recv, ccw_send, ccw_recv):
        my = lax.axis_index("i")
        left = lax.rem(my + N_DEV - 1, N_DEV)
        right = lax.rem(my + 1, N_DEV)

        barrier = pltpu.get_barrier_semaphore()
        _sem_signal(barrier, inc=1, device_id=(left,), device_id_type=MESH)
        _sem_signal(barrier, inc=1, device_id=(right,), device_id_type=MESH)
        _sem_wait(barrier, 2)

        def cw_off(h, p):
            return lax.rem(my + N_DEV - h, N_DEV) * m_per + p * piece

        def ccw_off(h, p):
            return lax.rem(my + h, N_DEV) * m_per + half + p * piece

        def rdma(src_ref, off, send_sem, recv_sem, dev):
            return pltpu.make_async_remote_copy(
                src_ref=src_ref,
                dst_ref=xg_ref.at[pl.ds(off, piece), :],
                send_sem=send_sem,
                recv_sem=recv_sem,
                device_id=(dev,),
                device_id_type=MESH,
            )

        def cw(h, p):
            off = cw_off(h, p)
            src = (x_ref.at[pl.ds(p * piece, piece), :] if h == 0
                   else xg_ref.at[pl.ds(off, piece), :])
            return rdma(src, off, cw_send.at[h, p], cw_recv.at[h, p], right)

        def ccw(h, p):
            off = ccw_off(h, p)
            src = (x_ref.at[pl.ds(half + p * piece, piece), :] if h == 0
                   else xg_ref.at[pl.ds(off, piece), :])
            return rdma(src, off, ccw_send.at[h, p], ccw_recv.at[h, p], left)

        def gemm(src_ref, row0, nrows):
            out0 = row0 if src_ref is xg_ref else my * m_per + row0
            acc = lax.dot_general(
                src_ref[pl.ds(row0, nrows), :],
                w_ref[...],
                dimension_numbers=(((1,), (0,)), ((), ())),
                preferred_element_type=jnp.float32,
            )
            out_ref[pl.ds(out0, nrows), :] = jnp.maximum(acc * s_ref[0], 0.0)

        for p in range(N_PIECE):
            cw(0, p).start()
            ccw(0, p).start()

        for h in range(1, N_DEV - 1):
            for p in range(N_PIECE):
                cw(h - 1, p).wait_recv()
                cw(h, p).start()
                ccw(h - 1, p).wait_recv()
                ccw(h, p).start()
                if p == 1:
                    if h == 1:
                        gemm(x_ref, 0, half)
                    else:
                        gemm(xg_ref, lax.rem(my + N_DEV - (h - 1), N_DEV) * m_per,
                             half)
                elif p == 3:
                    if h == 1:
                        gemm(x_ref, half, half)
                    else:
                        gemm(xg_ref, lax.rem(my + h - 1, N_DEV) * m_per + half,
                             half)
            for p in range(N_PIECE):
                cw(h - 1, p).wait_send()
                ccw(h - 1, p).wait_send()

        last = N_DEV - 2
        gemm(xg_ref, lax.rem(my + N_DEV - last, N_DEV) * m_per, half)
        gemm(xg_ref, lax.rem(my + last, N_DEV) * m_per + half, half)
        for p in range(N_PIECE):
            cw(last, p).wait_recv()
            ccw(last, p).wait_recv()
            cw(last, p).wait_send()
            ccw(last, p).wait_send()
        gemm(xg_ref, lax.rem(my + 1, N_DEV) * m_per, half)
        gemm(xg_ref, lax.rem(my + N_DEV - 1, N_DEV) * m_per + half, half)

    return pl.pallas_call(
        body,
        out_shape=jax.ShapeDtypeStruct((N_DEV * m_per, n_per), jnp.float32),
        in_specs=[
            pl.BlockSpec(memory_space=pltpu.VMEM),
            pl.BlockSpec(memory_space=pltpu.VMEM),
            pl.BlockSpec(memory_space=pltpu.SMEM),
        ],
        out_specs=pl.BlockSpec(memory_space=pltpu.VMEM),
        scratch_shapes=[
            pltpu.VMEM((N_DEV * m_per, k), jnp.float8_e4m3fn),
            pltpu.SemaphoreType.DMA((N_DEV - 1, N_PIECE)),
            pltpu.SemaphoreType.DMA((N_DEV - 1, N_PIECE)),
            pltpu.SemaphoreType.DMA((N_DEV - 1, N_PIECE)),
            pltpu.SemaphoreType.DMA((N_DEV - 1, N_PIECE)),
        ],
        compiler_params=_CompilerParams(
            collective_id=0, vmem_limit_bytes=60 * 1024 * 1024
        ),
    )(x8, w8, scale)


# device time: 86458 ns/iter; 1.2051x vs baseline; 1.2051x over previous
import jax
import jax.numpy as jnp
from jax import lax
from jax.experimental import pallas as pl
from jax.experimental.pallas import tpu as pltpu

N_DEV = 8
GENS = (1, 3, 4)
PERM = ((1, 3, 4), (3, 4, 1), (4, 1, 3))
MASKS = tuple(
    ({0: (0,), 1: (0, p[0]), 2: (0, p[0], p[1], p[0] ^ p[1])}) for p in PERM
)

_DeviceIdType = getattr(pl, "DeviceIdType", None) or pltpu.DeviceIdType
MESH = _DeviceIdType.MESH
_sem_signal = getattr(pl, "semaphore_signal", None) or pltpu.semaphore_signal
_sem_wait = getattr(pl, "semaphore_wait", None) or pltpu.semaphore_wait
_CompilerParams = getattr(pltpu, "CompilerParams", None) or pltpu.TPUCompilerParams


def kernel(x, w_mat, scale_x, scale_w):
    m_per, k = x.shape
    _, n_per = w_mat.shape
    rs_off = (0, 160, 320)
    rs_len = (160, 160, 192)
    assert rs_off[2] + rs_len[2] == m_per

    x8 = x.astype(jnp.float8_e4m3fn)
    w8 = w_mat.astype(jnp.float8_e5m2)
    scale = (scale_x.reshape(-1)[:1] * scale_w.reshape(-1)[:1]).astype(jnp.float32)

    def body(x_ref, w_ref, s_ref, out_ref, xg_ref, send_sems, recv_sems):
        my = lax.axis_index("i")

        barrier = pltpu.get_barrier_semaphore()
        for g in GENS:
            _sem_signal(barrier, inc=1, device_id=(my ^ g,), device_id_type=MESH)
        _sem_wait(barrier, len(GENS))

        def gemm(src_ref, row0, nrows, out0=None):
            acc = lax.dot_general(
                src_ref[pl.ds(row0, nrows), :],
                w_ref[...],
                dimension_numbers=(((1,), (0,)), ((), ())),
                preferred_element_type=jnp.float32,
            )
            out0 = row0 if out0 is None else out0
            out_ref[pl.ds(out0, nrows), :] = jnp.maximum(acc * s_ref[0], 0.0)

        def part_rdma(s, r, j, m):
            g = PERM[s][r]
            row0 = (my ^ m) * m_per + rs_off[s]
            if m == 0:
                src = x_ref.at[pl.ds(rs_off[s], rs_len[s]), :]
            else:
                src = xg_ref.at[pl.ds(row0, rs_len[s]), :]
            return pltpu.make_async_remote_copy(
                src_ref=src,
                dst_ref=xg_ref.at[pl.ds(row0, rs_len[s]), :],
                send_sem=send_sems.at[s, r, j],
                recv_sem=recv_sems.at[s, r, j],
                device_id=(my ^ g,),
                device_id_type=MESH,
            )

        def recv_rdma(s, r, j, m):
            g = PERM[s][r]
            row0 = (my ^ g ^ m) * m_per + rs_off[s]
            ref = xg_ref.at[pl.ds(row0, rs_len[s]), :]
            return pltpu.make_async_remote_copy(
                src_ref=ref,
                dst_ref=ref,
                send_sem=send_sems.at[s, r, j],
                recv_sem=recv_sems.at[s, r, j],
                device_id=(my ^ g,),
                device_id_type=MESH,
            )

        for s in range(3):
            part_rdma(s, 0, 0, 0).start()
        gemm(x_ref, 0, m_per, out0=my * m_per)

        for r in range(3):
            for s in range(3):
                g = PERM[s][r]
                for j, m in enumerate(MASKS[s][r]):
                    recv_rdma(s, r, j, m).wait_recv()
                    if r == 2:
                        gemm(xg_ref, (my ^ g ^ m) * m_per + rs_off[s], rs_len[s])
                if r < 2:
                    for j, m in enumerate(MASKS[s][r + 1]):
                        part_rdma(s, r + 1, j, m).start()
            if r < 2:
                for s in range(3):
                    g = PERM[s][r]
                    for m in MASKS[s][r]:
                        gemm(xg_ref, (my ^ g ^ m) * m_per + rs_off[s], rs_len[s])

        for s in range(3):
            for r in range(3):
                for j, m in enumerate(MASKS[s][r]):
                    part_rdma(s, r, j, m).wait_send()

    return pl.pallas_call(
        body,
        out_shape=jax.ShapeDtypeStruct((N_DEV * m_per, n_per), jnp.float32),
        in_specs=[
            pl.BlockSpec(memory_space=pltpu.VMEM),
            pl.BlockSpec(memory_space=pltpu.VMEM),
            pl.BlockSpec(memory_space=pltpu.SMEM),
        ],
        out_specs=pl.BlockSpec(memory_space=pltpu.VMEM),
        scratch_shapes=[
            pltpu.VMEM((N_DEV * m_per, k), jnp.float8_e4m3fn),
            pltpu.SemaphoreType.DMA((3, 3, 4)),
            pltpu.SemaphoreType.DMA((3, 3, 4)),
        ],
        compiler_params=_CompilerParams(
            collective_id=0, vmem_limit_bytes=60 * 1024 * 1024
        ),
    )(x8, w8, scale)


# device time: 80661 ns/iter; 1.2917x vs baseline; 1.0719x over previous
import jax
import jax.numpy as jnp
from jax import lax
from jax.experimental import pallas as pl
from jax.experimental.pallas import tpu as pltpu

N_DEV = 8
PERM = ((1, 3, 4), (3, 4, 1), (4, 1, 3))
MASKS = tuple(
    {0: (0,), 1: (0, p[0]), 2: (0, p[0], p[1], p[0] ^ p[1])} for p in PERM
)
FWD = {(0, 0): ((1, 1), (2, 1)), (1, 0): ((2, 2),), (1, 1): ((2, 3),)}
ORDER = [
    (0, 0, 0), (0, 1, 0), (0, 2, 0),
    (1, 0, 0), (1, 1, 0), (1, 2, 0),
    (2, 0, 0), (2, 1, 0), (2, 2, 0),
    (0, 1, 1), (0, 2, 1), (1, 2, 1),
    (1, 2, 2), (0, 2, 2),
    (2, 1, 1), (2, 2, 2),
    (1, 1, 1),
    (2, 2, 1),
    (0, 2, 3),
    (1, 2, 3),
    (2, 2, 3),
]

_DeviceIdType = getattr(pl, "DeviceIdType", None) or pltpu.DeviceIdType
MESH = _DeviceIdType.MESH
_sem_signal = getattr(pl, "semaphore_signal", None) or pltpu.semaphore_signal
_sem_wait = getattr(pl, "semaphore_wait", None) or pltpu.semaphore_wait
_CompilerParams = getattr(pltpu, "CompilerParams", None) or pltpu.TPUCompilerParams


def kernel(x, w_mat, scale_x, scale_w):
    m_per, k = x.shape
    _, n_per = w_mat.shape
    rs_off = (0, 160, 320)
    rs_len = (160, 160, 192)
    assert rs_off[2] + rs_len[2] == m_per

    x8 = x.astype(jnp.float8_e4m3fn)
    w8 = w_mat.astype(jnp.float8_e5m2)
    scale = (scale_x.reshape(-1)[:1] * scale_w.reshape(-1)[:1]).astype(jnp.float32)

    def body(x_ref, w_ref, s_ref, out_ref, xg_ref, send_sems, recv_sems):
        my = lax.axis_index("i")

        barrier = pltpu.get_barrier_semaphore()
        for g in (1, 3, 4):
            _sem_signal(barrier, inc=1, device_id=(my ^ g,), device_id_type=MESH)
        _sem_wait(barrier, 3)

        def gemm(src_ref, row0, nrows, out0=None):
            acc = lax.dot_general(
                src_ref[pl.ds(row0, nrows), :],
                w_ref[...],
                dimension_numbers=(((1,), (0,)), ((), ())),
                preferred_element_type=jnp.float32,
            )
            out0 = row0 if out0 is None else out0
            out_ref[pl.ds(out0, nrows), :] = jnp.maximum(acc * s_ref[0], 0.0)

        def part_rdma(s, r, j):
            g = PERM[s][r]
            m = MASKS[s][r][j]
            row0 = (my ^ m) * m_per + rs_off[s]
            if m == 0:
                src = x_ref.at[pl.ds(rs_off[s], rs_len[s]), :]
            else:
                src = xg_ref.at[pl.ds(row0, rs_len[s]), :]
            return pltpu.make_async_remote_copy(
                src_ref=src,
                dst_ref=xg_ref.at[pl.ds(row0, rs_len[s]), :],
                send_sem=send_sems.at[s, r, j],
                recv_sem=recv_sems.at[s, r, j],
                device_id=(my ^ g,),
                device_id_type=MESH,
            )

        def recv_wait_and_process(s, r, j):
            g = PERM[s][r]
            m = MASKS[s][r][j]
            row0 = (my ^ g ^ m) * m_per + rs_off[s]
            ref = xg_ref.at[pl.ds(row0, rs_len[s]), :]
            desc = pltpu.make_async_remote_copy(
                src_ref=ref,
                dst_ref=ref,
                send_sem=send_sems.at[s, r, j],
                recv_sem=recv_sems.at[s, r, j],
                device_id=(my ^ g,),
                device_id_type=MESH,
            )
            desc.wait_recv()
            for fr, fj in FWD.get((r, j), ()):
                part_rdma(s, fr, fj).start()
            gemm(xg_ref, row0, rs_len[s])

        for s in range(3):
            for r in range(3):
                part_rdma(s, r, 0).start()
        gemm(x_ref, 0, m_per, out0=my * m_per)

        for s, r, j in ORDER:
            recv_wait_and_process(s, r, j)

        for s in range(3):
            for r in range(3):
                for j in range(len(MASKS[s][r])):
                    part_rdma(s, r, j).wait_send()

    return pl.pallas_call(
        body,
        out_shape=jax.ShapeDtypeStruct((N_DEV * m_per, n_per), jnp.float32),
        in_specs=[
            pl.BlockSpec(memory_space=pltpu.VMEM),
            pl.BlockSpec(memory_space=pltpu.VMEM),
            pl.BlockSpec(memory_space=pltpu.SMEM),
        ],
        out_specs=pl.BlockSpec(memory_space=pltpu.VMEM),
        scratch_shapes=[
            pltpu.VMEM((N_DEV * m_per, k), jnp.float8_e4m3fn),
            pltpu.SemaphoreType.DMA((3, 3, 4)),
            pltpu.SemaphoreType.DMA((3, 3, 4)),
        ],
        compiler_params=_CompilerParams(
            collective_id=0, vmem_limit_bytes=60 * 1024 * 1024
        ),
    )(x8, w8, scale)
